# Initial kernel scaffold; baseline (speedup 1.0000x reference)
import jax
import jax.numpy as jnp
from jax.experimental import pallas as pl


def kernel(graph, input, W1, b1, W2, b2, pe_coff, pe, t):
    raise NotImplementedError("write your pallas kernel here")


# speedup vs baseline: 1.3802x; 1.3802x over previous
import jax
import jax.numpy as jnp
from jax.experimental import pallas as pl


def _mlp_gate_block(x_ref, pe_ref, w1_ref, b1_ref, w2_ref, b2_ref, o_ref):
    x = x_ref[...] + pe_ref[...]
    h = jnp.dot(x, w1_ref[...], preferred_element_type=jnp.float32)
    h = jnp.maximum(h + b1_ref[...], 0.0)
    m = jnp.dot(h, w2_ref[...], preferred_element_type=jnp.float32)
    m = jnp.maximum(m + b2_ref[...], 0.0)
    o_ref[...] = x * jax.nn.sigmoid(m)


def kernel(graph, input, W1, b1, W2, b2, pe_coff, pe, t):
    n, d = input.shape
    pe_row = pe_coff * jax.lax.dynamic_index_in_dim(pe, t + 1, axis=0, keepdims=True)
    w2_col = W2[:, 1:2]
    b2_col = b2[1].reshape(1, 1)
    b1_row = b1.reshape(1, d)

    block = 4000
    assert n % block == 0
    grid = (n // block,)

    return pl.pallas_call(
        _mlp_gate_block,
        grid=grid,
        in_specs=[
            pl.BlockSpec((block, d), lambda i: (i, 0)),
            pl.BlockSpec((1, d), lambda i: (0, 0)),
            pl.BlockSpec((d, d), lambda i: (0, 0)),
            pl.BlockSpec((1, d), lambda i: (0, 0)),
            pl.BlockSpec((d, 1), lambda i: (0, 0)),
            pl.BlockSpec((1, 1), lambda i: (0, 0)),
        ],
        out_specs=pl.BlockSpec((block, d), lambda i: (i, 0)),
        out_shape=jax.ShapeDtypeStruct((n, d), jnp.float32),
    )(input, pe_row, W1, b1_row, w2_col, b2_col)


# speedup vs baseline: 1.6630x; 1.2049x over previous
import jax
import jax.numpy as jnp
from jax.experimental import pallas as pl


def _mlp_gate_block(x_ref, pe_ref, w1_ref, b1_ref, w2_ref, b2_ref, o_ref):
    x = x_ref[...] + pe_ref[...]
    h = jnp.dot(x, w1_ref[...], preferred_element_type=jnp.float32)
    h = jnp.maximum(h + b1_ref[...], 0.0)
    m = jnp.dot(h, w2_ref[...], preferred_element_type=jnp.float32)
    m = jnp.maximum(m + b2_ref[...], 0.0)
    o_ref[...] = x * jax.nn.sigmoid(m)


def kernel(graph, input, W1, b1, W2, b2, pe_coff, pe, t):
    n, d = input.shape
    pe_row = pe_coff * jax.lax.dynamic_index_in_dim(pe, t + 1, axis=0, keepdims=True)
    w2_col = W2[:, 1:2]
    b2_col = b2[1].reshape(1, 1)
    b1_row = b1.reshape(1, d)

    block = 10000
    assert n % block == 0
    grid = (n // block,)

    return pl.pallas_call(
        _mlp_gate_block,
        grid=grid,
        in_specs=[
            pl.BlockSpec((block, d), lambda i: (i, 0)),
            pl.BlockSpec((1, d), lambda i: (0, 0)),
            pl.BlockSpec((d, d), lambda i: (0, 0)),
            pl.BlockSpec((1, d), lambda i: (0, 0)),
            pl.BlockSpec((d, 1), lambda i: (0, 0)),
            pl.BlockSpec((1, 1), lambda i: (0, 0)),
        ],
        out_specs=pl.BlockSpec((block, d), lambda i: (i, 0)),
        out_shape=jax.ShapeDtypeStruct((n, d), jnp.float32),
    )(input, pe_row, W1, b1_row, w2_col, b2_col)


# speedup vs baseline: 1.6740x; 1.0066x over previous
import jax
import jax.numpy as jnp
from jax.experimental import pallas as pl


def _mlp_gate_block(x_ref, pe_ref, w1_ref, b1_ref, w2_ref, b2_ref, o_ref):
    x = x_ref[...] + pe_ref[...]
    h = jnp.dot(x, w1_ref[...], preferred_element_type=jnp.float32)
    h = jnp.maximum(h + b1_ref[...], 0.0)
    m = jnp.dot(h, w2_ref[...], preferred_element_type=jnp.float32)
    m = jnp.maximum(m + b2_ref[...], 0.0)
    o_ref[...] = x * jax.nn.sigmoid(m)


def kernel(graph, input, W1, b1, W2, b2, pe_coff, pe, t):
    n, d = input.shape
    pe_row = pe_coff * jax.lax.dynamic_index_in_dim(pe, t + 1, axis=0, keepdims=True)
    w2_col = W2[:, 1:2]
    b2_col = b2[1].reshape(1, 1)
    b1_row = b1.reshape(1, d)

    block = 20000
    assert n % block == 0
    grid = (n // block,)

    return pl.pallas_call(
        _mlp_gate_block,
        grid=grid,
        in_specs=[
            pl.BlockSpec((block, d), lambda i: (i, 0)),
            pl.BlockSpec((1, d), lambda i: (0, 0)),
            pl.BlockSpec((d, d), lambda i: (0, 0)),
            pl.BlockSpec((1, d), lambda i: (0, 0)),
            pl.BlockSpec((d, 1), lambda i: (0, 0)),
            pl.BlockSpec((1, 1), lambda i: (0, 0)),
        ],
        out_specs=pl.BlockSpec((block, d), lambda i: (i, 0)),
        out_shape=jax.ShapeDtypeStruct((n, d), jnp.float32),
    )(input, pe_row, W1, b1_row, w2_col, b2_col)
